# baseline (speedup 1.0000x reference)
import functools

import jax
import jax.numpy as jnp
from jax import lax
from jax.experimental import pallas as pl
from jax.experimental.pallas import tpu as pltpu
from jax.experimental.pallas import tpu_sc as plsc

N = 10000
E = 320000
D = 128
NC = 2
NS = 16
NW = NC * NS
L = 16
_SC_PARAMS = pltpu.CompilerParams(
    needs_layout_passes=False, use_tc_tiling_on_sc=False)
EC = E // NW
NBLK = EC // L
RPT = N // NS


def _lin_body(x_ref, wt_ref, b_ref, wa_ref, ba_ref, h_ref, s1_ref, s2_ref):
    h = jnp.dot(x_ref[...], wt_ref[...], preferred_element_type=jnp.float32)
    h = h + b_ref[...]
    h_ref[...] = h
    cdims = (((1,), (1,)), ((), ()))
    s1 = jax.lax.dot_general(wa_ref[0:1], h, cdims,
                             preferred_element_type=jnp.float32)
    s2 = jax.lax.dot_general(wa_ref[1:2], h, cdims,
                             preferred_element_type=jnp.float32)
    s1_ref[0] = s1 + ba_ref[0]
    s2_ref[0] = s2


def _linear(x, wt, b, wa, ba):
    blk = 1000
    grid = N // blk
    return pl.pallas_call(
        _lin_body,
        grid=(grid,),
        in_specs=[
            pl.BlockSpec((blk, D), lambda i: (i, 0)),
            pl.BlockSpec((D, D), lambda i: (0, 0)),
            pl.BlockSpec((1, D), lambda i: (0, 0)),
            pl.BlockSpec((2, D), lambda i: (0, 0)),
            pl.BlockSpec(memory_space=pltpu.SMEM),
        ],
        out_specs=[
            pl.BlockSpec((blk, D), lambda i: (i, 0)),
            pl.BlockSpec((1, 1, blk), lambda i: (i, 0, 0)),
            pl.BlockSpec((1, 1, blk), lambda i: (i, 0, 0)),
        ],
        out_shape=[
            jax.ShapeDtypeStruct((N, D), jnp.float32),
            jax.ShapeDtypeStruct((grid, 1, blk), jnp.float32),
            jax.ShapeDtypeStruct((grid, 1, blk), jnp.float32),
        ],
    )(x, wt, b, wa, ba)


def _edge_scalar_body(ei_hbm, s1_hbm, s2_hbm,
                      p_hbm, zpart_hbm, degpart_hbm,
                      rows_v, cols_v, s1_v, s2_v, p_v, z_v, deg_v):
    cid = lax.axis_index("c")
    sid = lax.axis_index("s")
    wid = sid * NC + cid
    base = wid * EC
    pltpu.sync_copy(ei_hbm.at[0, pl.ds(base, EC)], rows_v)
    pltpu.sync_copy(ei_hbm.at[1, pl.ds(base, EC)], cols_v)
    pltpu.sync_copy(s1_hbm, s1_v)
    pltpu.sync_copy(s2_hbm, s2_v)

    zeros = jnp.zeros((L,), jnp.float32)

    def zinit(i, carry):
        for u in range(5):
            z_v[pl.ds((i * 5 + u) * L, L)] = zeros
            deg_v[pl.ds((i * 5 + u) * L, L)] = zeros
        return carry

    lax.fori_loop(0, N // (5 * L), zinit, 0)

    ones = jnp.ones((L,), jnp.float32)

    def ebody(i, carry):
        for u in range(5):
            k = (i * 5 + u) * L
            r = rows_v[pl.ds(k, L)]
            c = cols_v[pl.ds(k, L)]
            a = plsc.load_gather(s1_v, [r]) + plsc.load_gather(s2_v, [c])
            e = jnp.maximum(a, a * 0.2)
            p = jnp.exp(e)
            p_v[pl.ds(k, L)] = p
            plsc.addupdate_scatter(z_v, [r], p)
            plsc.addupdate_scatter(deg_v, [c], ones)
        return carry

    lax.fori_loop(0, NBLK // 5, ebody, 0)

    pltpu.sync_copy(p_v, p_hbm.at[pl.ds(base, EC)])

    def wpart(jj, carry):
        pltpu.sync_copy(z_v.at[pl.ds(jj * ZB, ZB)], zpart_hbm.at[jj, wid])
        pltpu.sync_copy(deg_v.at[pl.ds(jj * ZB, ZB)], degpart_hbm.at[jj, wid])
        return carry

    lax.fori_loop(0, N // ZB, wpart, 0)


def _edge_scalar(ei, s1, s2):
    mesh = plsc.VectorSubcoreMesh(
        core_axis_name="c", subcore_axis_name="s", num_cores=NC, num_subcores=NS)
    fn = pl.kernel(
        _edge_scalar_body,
        out_type=[
            jax.ShapeDtypeStruct((E,), jnp.float32),
            jax.ShapeDtypeStruct((N // ZB, NW, ZB), jnp.float32),
            jax.ShapeDtypeStruct((N // ZB, NW, ZB), jnp.float32),
        ],
        mesh=mesh,
        scratch_types=[
            pltpu.VMEM((EC,), jnp.int32),
            pltpu.VMEM((EC,), jnp.int32),
            pltpu.VMEM((N,), jnp.float32),
            pltpu.VMEM((N,), jnp.float32),
            pltpu.VMEM((EC,), jnp.float32),
            pltpu.VMEM((N,), jnp.float32),
            pltpu.VMEM((N,), jnp.float32),
        ],
        compiler_params=_SC_PARAMS,
    )
    return fn(ei, s1, s2)


def _col_sums(m):
    ones = jnp.ones((m.shape[0], 1), jnp.float32)
    return jax.lax.dot_general(m, ones, (((0,), (0,)), ((), ())),
                               preferred_element_type=jnp.float32)


def _zr_body(zp_ref, zr_ref):
    ones = jnp.ones((1, NW), jnp.float32)
    z = jax.lax.dot_general(ones, zp_ref[0], (((1,), (0,)), ((), ())),
                            preferred_element_type=jnp.float32)
    zr_ref[0] = 1.0 / (z + 1e-16)


def _recip_z(zpart3):
    blk = 1000
    return pl.pallas_call(
        _zr_body,
        grid=(N // blk,),
        in_specs=[
            pl.BlockSpec((1, NW, blk), lambda i: (i, 0, 0)),
        ],
        out_specs=pl.BlockSpec((1, 1, blk), lambda i: (i, 0, 0)),
        out_shape=jax.ShapeDtypeStruct((N // blk, 1, blk), jnp.float32),
    )(zpart3)


ZB = 1000


NB = 5
PF = 4


def _agg_body(ei_hbm, p_hbm, h_hbm, zr_hbm, zeros_hbm, acc_hbm,
              rows_v, cols_v, p_v, zr_v, gbuf, acc_sh,
              g0, g1, g2, g3, g4, s0, s1, s2, s3, s4):
    gsems = (g0, g1, g2, g3, g4)
    ssems = (s0, s1, s2, s3, s4)
    cid = lax.axis_index("c")
    sid = lax.axis_index("s")
    wid = sid * NC + cid
    base = wid * EC
    d1 = pltpu.async_copy(ei_hbm.at[0, pl.ds(base, EC)], rows_v, g0)
    d2 = pltpu.async_copy(ei_hbm.at[1, pl.ds(base, EC)], cols_v, g1)
    d3 = pltpu.async_copy(p_hbm.at[pl.ds(base, EC)], p_v, g2)
    d4 = pltpu.async_copy(zr_hbm, zr_v, g3)
    d5 = pltpu.async_copy(zeros_hbm, acc_sh.at[pl.ds(sid * RPT, RPT)], g4)
    d1.wait(); d2.wait(); d3.wait(); d4.wait(); d5.wait()
    plsc.subcore_barrier()

    def g_start(i, b):
        r = rows_v[pl.ds(i * L, L)]
        pltpu.async_copy(h_hbm.at[r], gbuf.at[b], gsems[b])

    def g_wait(b):
        pltpu.make_async_copy(h_hbm.at[pl.ds(0, L)], gbuf.at[b], gsems[b]).wait()

    def s_start(i, b):
        c = cols_v[pl.ds(i * L, L)]
        pltpu.async_copy(gbuf.at[b], acc_sh.at[c], ssems[b], add=True)

    def s_wait(b):
        pltpu.make_async_copy(h_hbm.at[pl.ds(0, L)], gbuf.at[b], ssems[b]).wait()

    _bidx = [jnp.full((L,), e, jnp.int32) for e in range(L)]

    def scale(i, b):
        r = rows_v[pl.ds(i * L, L)]
        pz = p_v[pl.ds(i * L, L)] * plsc.load_gather(zr_v, [r])
        for e in range(L):
            pb = jnp.take_along_axis(pz, _bidx[e], axis=0,
                                     mode="promise_in_bounds")
            for j in range(D // L):
                gbuf[b, e, pl.ds(j * L, L)] = gbuf[b, e, pl.ds(j * L, L)] * pb

    for b in range(PF):
        g_start(b, b)

    n_outer = NBLK // NB

    def outer(io, carry):
        for b in range(NB):
            i = io * NB + b
            nxt = i + PF
            bb = (b + PF) % NB
            if b < NB - PF:
                if b < 2:
                    @pl.when(io > 0)
                    def _():
                        s_wait(bb)
                else:
                    s_wait(bb)
                g_start(nxt, bb)
            else:
                @pl.when(io < n_outer - 1)
                def _():
                    s_wait(bb)
                    g_start(nxt, bb)
            g_wait(b)
            scale(i, b)
            s_start(i, b)
        return carry

    lax.fori_loop(0, n_outer, outer, 0)
    for b in range(NB):
        s_wait(b)
    plsc.subcore_barrier()

    pltpu.sync_copy(acc_sh.at[pl.ds(sid * RPT, RPT)],
                    acc_hbm.at[cid, pl.ds(sid * RPT, RPT)])


def _aggregate(ei, p, h, zr):
    mesh = plsc.VectorSubcoreMesh(
        core_axis_name="c", subcore_axis_name="s", num_cores=NC, num_subcores=NS)
    fn = pl.kernel(
        _agg_body,
        out_type=jax.ShapeDtypeStruct((NC, N, D), jnp.float32),
        mesh=mesh,
        scratch_types=[
            pltpu.VMEM((EC,), jnp.int32),
            pltpu.VMEM((EC,), jnp.int32),
            pltpu.VMEM((EC,), jnp.float32),
            pltpu.VMEM((N,), jnp.float32),
            pltpu.VMEM((NB, L, D), jnp.float32),
            pltpu.VMEM_SHARED((N, D), jnp.float32),
        ] + [pltpu.SemaphoreType.DMA] * (2 * NB),
        compiler_params=_SC_PARAMS,
    )
    return fn(ei, p, h, zr, jnp.zeros((RPT, D), jnp.float32))


def _combine_body(acc_ref, dp_ref, h_ref, o_ref):
    deg = _col_sums(dp_ref[0])
    o_ref[...] = jnp.where(deg > 0.0, acc_ref[0] + acc_ref[1], h_ref[...])


def _combine(acc, dp3, h):
    blk = 1000
    return pl.pallas_call(
        _combine_body,
        grid=(N // blk,),
        in_specs=[
            pl.BlockSpec((NC, blk, D), lambda i: (0, i, 0)),
            pl.BlockSpec((1, NW, blk), lambda i: (i, 0, 0)),
            pl.BlockSpec((blk, D), lambda i: (i, 0)),
        ],
        out_specs=pl.BlockSpec((blk, D), lambda i: (i, 0)),
        out_shape=jax.ShapeDtypeStruct((N, D), jnp.float32),
    )(acc, dp3, h)


def kernel(x, edge_index, W_lin, b_lin, W_att, b_att):
    wt = W_lin.T
    b = b_lin.reshape(1, D)
    wa = W_att.reshape(2, D)

    h, s1_3, s2_3 = _linear(x, wt, b, wa, b_att)
    s1 = s1_3.reshape(N)
    s2 = s2_3.reshape(N)

    p, zpart3, degpart3 = _edge_scalar(edge_index, s1, s2)
    zr = _recip_z(zpart3).reshape(N)
    acc = _aggregate(edge_index, p, h, zr)
    return _combine(acc, degpart3, h)


# speedup vs baseline: 1.0475x; 1.0475x over previous
import functools

import jax
import jax.numpy as jnp
from jax import lax
from jax.experimental import pallas as pl
from jax.experimental.pallas import tpu as pltpu
from jax.experimental.pallas import tpu_sc as plsc

N = 10000
E = 320000
D = 128
NC = 2
NS = 16
NW = NC * NS
L = 16
_SC_PARAMS = pltpu.CompilerParams(
    needs_layout_passes=False, use_tc_tiling_on_sc=False)
EC = E // NW
NBLK = EC // L
RPT = N // NS


def _lin_body(x_ref, wt_ref, b_ref, wa_ref, ba_ref, h_ref, s1_ref, s2_ref):
    h = jnp.dot(x_ref[...], wt_ref[...], preferred_element_type=jnp.float32)
    h = h + b_ref[...]
    h_ref[...] = h
    cdims = (((1,), (1,)), ((), ()))
    s1 = jax.lax.dot_general(wa_ref[0:1], h, cdims,
                             preferred_element_type=jnp.float32)
    s2 = jax.lax.dot_general(wa_ref[1:2], h, cdims,
                             preferred_element_type=jnp.float32)
    s1_ref[0] = s1 + ba_ref[0]
    s2_ref[0] = s2


def _linear(x, wt, b, wa, ba):
    blk = 1000
    grid = N // blk
    return pl.pallas_call(
        _lin_body,
        grid=(grid,),
        in_specs=[
            pl.BlockSpec((blk, D), lambda i: (i, 0)),
            pl.BlockSpec((D, D), lambda i: (0, 0)),
            pl.BlockSpec((1, D), lambda i: (0, 0)),
            pl.BlockSpec((2, D), lambda i: (0, 0)),
            pl.BlockSpec(memory_space=pltpu.SMEM),
        ],
        out_specs=[
            pl.BlockSpec((blk, D), lambda i: (i, 0)),
            pl.BlockSpec((1, 1, blk), lambda i: (i, 0, 0)),
            pl.BlockSpec((1, 1, blk), lambda i: (i, 0, 0)),
        ],
        out_shape=[
            jax.ShapeDtypeStruct((N, D), jnp.float32),
            jax.ShapeDtypeStruct((grid, 1, blk), jnp.float32),
            jax.ShapeDtypeStruct((grid, 1, blk), jnp.float32),
        ],
    )(x, wt, b, wa, ba)


def _edge_scalar_body(ei_hbm, s1_hbm, s2_hbm,
                      p_hbm, zpart_hbm, degpart_hbm,
                      rows_v, cols_v, s1_v, s2_v, p_v, z_v, deg_v):
    cid = lax.axis_index("c")
    sid = lax.axis_index("s")
    wid = sid * NC + cid
    base = wid * EC
    pltpu.sync_copy(ei_hbm.at[0, pl.ds(base, EC)], rows_v)
    pltpu.sync_copy(ei_hbm.at[1, pl.ds(base, EC)], cols_v)
    pltpu.sync_copy(s1_hbm, s1_v)
    pltpu.sync_copy(s2_hbm, s2_v)

    zeros = jnp.zeros((L,), jnp.float32)

    def zinit(i, carry):
        for u in range(5):
            z_v[pl.ds((i * 5 + u) * L, L)] = zeros
            deg_v[pl.ds((i * 5 + u) * L, L)] = zeros
        return carry

    lax.fori_loop(0, N // (5 * L), zinit, 0)

    ones = jnp.ones((L,), jnp.float32)

    def ebody(i, carry):
        for u in range(5):
            k = (i * 5 + u) * L
            r = rows_v[pl.ds(k, L)]
            c = cols_v[pl.ds(k, L)]
            a = plsc.load_gather(s1_v, [r]) + plsc.load_gather(s2_v, [c])
            e = jnp.maximum(a, a * 0.2)
            p = jnp.exp(e)
            p_v[pl.ds(k, L)] = p
            plsc.addupdate_scatter(z_v, [r], p)
            plsc.addupdate_scatter(deg_v, [c], ones)
        return carry

    lax.fori_loop(0, NBLK // 5, ebody, 0)

    pltpu.sync_copy(p_v, p_hbm.at[pl.ds(base, EC)])

    def wpart(jj, carry):
        pltpu.sync_copy(z_v.at[pl.ds(jj * ZB, ZB)], zpart_hbm.at[jj, wid])
        pltpu.sync_copy(deg_v.at[pl.ds(jj * ZB, ZB)], degpart_hbm.at[jj, wid])
        return carry

    lax.fori_loop(0, N // ZB, wpart, 0)


def _edge_scalar(ei, s1, s2):
    mesh = plsc.VectorSubcoreMesh(
        core_axis_name="c", subcore_axis_name="s", num_cores=NC, num_subcores=NS)
    fn = pl.kernel(
        _edge_scalar_body,
        out_type=[
            jax.ShapeDtypeStruct((E,), jnp.float32),
            jax.ShapeDtypeStruct((N // ZB, NW, ZB), jnp.float32),
            jax.ShapeDtypeStruct((N // ZB, NW, ZB), jnp.float32),
        ],
        mesh=mesh,
        scratch_types=[
            pltpu.VMEM((EC,), jnp.int32),
            pltpu.VMEM((EC,), jnp.int32),
            pltpu.VMEM((N,), jnp.float32),
            pltpu.VMEM((N,), jnp.float32),
            pltpu.VMEM((EC,), jnp.float32),
            pltpu.VMEM((N,), jnp.float32),
            pltpu.VMEM((N,), jnp.float32),
        ],
        compiler_params=_SC_PARAMS,
    )
    return fn(ei, s1, s2)


def _col_sums(m):
    ones = jnp.ones((m.shape[0], 1), jnp.float32)
    return jax.lax.dot_general(m, ones, (((0,), (0,)), ((), ())),
                               preferred_element_type=jnp.float32)


def _zr_body(zp_ref, zr_ref):
    ones = jnp.ones((1, NW), jnp.float32)
    z = jax.lax.dot_general(ones, zp_ref[0], (((1,), (0,)), ((), ())),
                            preferred_element_type=jnp.float32)
    zr_ref[0] = 1.0 / (z + 1e-16)


def _recip_z(zpart3):
    blk = 1000
    return pl.pallas_call(
        _zr_body,
        grid=(N // blk,),
        in_specs=[
            pl.BlockSpec((1, NW, blk), lambda i: (i, 0, 0)),
        ],
        out_specs=pl.BlockSpec((1, 1, blk), lambda i: (i, 0, 0)),
        out_shape=jax.ShapeDtypeStruct((N // blk, 1, blk), jnp.float32),
    )(zpart3)


ZB = 1000


NB = 5
PF = 3


def _agg_body(ei_hbm, p_hbm, h_hbm, zr_hbm, zeros_hbm, acc_hbm,
              rows_v, cols_v, p_v, zr_v, gbuf, acc_sh,
              g0, g1, g2, g3, g4, s0, s1, s2, s3, s4):
    gsems = (g0, g1, g2, g3, g4)
    ssems = (s0, s1, s2, s3, s4)
    cid = lax.axis_index("c")
    sid = lax.axis_index("s")
    wid = sid * NC + cid
    base = wid * EC
    d1 = pltpu.async_copy(ei_hbm.at[0, pl.ds(base, EC)], rows_v, g0)
    d2 = pltpu.async_copy(ei_hbm.at[1, pl.ds(base, EC)], cols_v, g1)
    d3 = pltpu.async_copy(p_hbm.at[pl.ds(base, EC)], p_v, g2)
    d4 = pltpu.async_copy(zr_hbm, zr_v, g3)
    d5 = pltpu.async_copy(zeros_hbm, acc_sh.at[pl.ds(sid * RPT, RPT)], g4)
    d1.wait(); d2.wait(); d3.wait(); d4.wait(); d5.wait()
    plsc.subcore_barrier()

    def g_start(i, b):
        r = rows_v[pl.ds(i * L, L)]
        pltpu.async_copy(h_hbm.at[r], gbuf.at[b], gsems[b])

    def g_wait(b):
        pltpu.make_async_copy(h_hbm.at[pl.ds(0, L)], gbuf.at[b], gsems[b]).wait()

    def s_start(i, b):
        c = cols_v[pl.ds(i * L, L)]
        pltpu.async_copy(gbuf.at[b], acc_sh.at[c], ssems[b], add=True)

    def s_wait(b):
        pltpu.make_async_copy(h_hbm.at[pl.ds(0, L)], gbuf.at[b], ssems[b]).wait()

    _bidx = [jnp.full((L,), e, jnp.int32) for e in range(L)]

    def scale(i, b):
        r = rows_v[pl.ds(i * L, L)]
        pz = p_v[pl.ds(i * L, L)] * plsc.load_gather(zr_v, [r])
        for e in range(L):
            pb = jnp.take_along_axis(pz, _bidx[e], axis=0,
                                     mode="promise_in_bounds")
            for j in range(D // L):
                gbuf[b, e, pl.ds(j * L, L)] = gbuf[b, e, pl.ds(j * L, L)] * pb

    for b in range(PF):
        g_start(b, b)

    n_outer = NBLK // NB

    def outer(io, carry):
        for b in range(NB):
            i = io * NB + b
            nxt = i + PF
            bb = (b + PF) % NB
            if b < NB - PF:
                if b < 2:
                    @pl.when(io > 0)
                    def _():
                        s_wait(bb)
                else:
                    s_wait(bb)
                g_start(nxt, bb)
            else:
                @pl.when(io < n_outer - 1)
                def _():
                    s_wait(bb)
                    g_start(nxt, bb)
            g_wait(b)
            scale(i, b)
            s_start(i, b)
        return carry

    lax.fori_loop(0, n_outer, outer, 0)
    for b in range(NB):
        s_wait(b)
    plsc.subcore_barrier()

    pltpu.sync_copy(acc_sh.at[pl.ds(sid * RPT, RPT)],
                    acc_hbm.at[cid, pl.ds(sid * RPT, RPT)])


def _aggregate(ei, p, h, zr):
    mesh = plsc.VectorSubcoreMesh(
        core_axis_name="c", subcore_axis_name="s", num_cores=NC, num_subcores=NS)
    fn = pl.kernel(
        _agg_body,
        out_type=jax.ShapeDtypeStruct((NC, N, D), jnp.float32),
        mesh=mesh,
        scratch_types=[
            pltpu.VMEM((EC,), jnp.int32),
            pltpu.VMEM((EC,), jnp.int32),
            pltpu.VMEM((EC,), jnp.float32),
            pltpu.VMEM((N,), jnp.float32),
            pltpu.VMEM((NB, L, D), jnp.float32),
            pltpu.VMEM_SHARED((N, D), jnp.float32),
        ] + [pltpu.SemaphoreType.DMA] * (2 * NB),
        compiler_params=_SC_PARAMS,
    )
    return fn(ei, p, h, zr, jnp.zeros((RPT, D), jnp.float32))


def _combine_body(acc_ref, dp_ref, h_ref, o_ref):
    deg = _col_sums(dp_ref[0])
    o_ref[...] = jnp.where(deg > 0.0, acc_ref[0] + acc_ref[1], h_ref[...])


def _combine(acc, dp3, h):
    blk = 1000
    return pl.pallas_call(
        _combine_body,
        grid=(N // blk,),
        in_specs=[
            pl.BlockSpec((NC, blk, D), lambda i: (0, i, 0)),
            pl.BlockSpec((1, NW, blk), lambda i: (i, 0, 0)),
            pl.BlockSpec((blk, D), lambda i: (i, 0)),
        ],
        out_specs=pl.BlockSpec((blk, D), lambda i: (i, 0)),
        out_shape=jax.ShapeDtypeStruct((N, D), jnp.float32),
    )(acc, dp3, h)


def kernel(x, edge_index, W_lin, b_lin, W_att, b_att):
    wt = W_lin.T
    b = b_lin.reshape(1, D)
    wa = W_att.reshape(2, D)

    h, s1_3, s2_3 = _linear(x, wt, b, wa, b_att)
    s1 = s1_3.reshape(N)
    s2 = s2_3.reshape(N)

    p, zpart3, degpart3 = _edge_scalar(edge_index, s1, s2)
    zr = _recip_z(zpart3).reshape(N)
    acc = _aggregate(edge_index, p, h, zr)
    return _combine(acc, degpart3, h)
